# baseline (speedup 1.0000x reference)
import jax
import jax.numpy as jnp
from jax.experimental import pallas as pl

_T = 2048
_HID = 256
_OUT_CH = 128
_NHEAD = 2
_HD = _HID // _NHEAD
_QBLK = 1024
_NEG = -1e30

_DNT = (((1,), (1,)), ((), ()))


def _bdott(a, b):
    return jax.lax.dot_general(a.astype(jnp.bfloat16), b.astype(jnp.bfloat16),
                               _DNT, preferred_element_type=jnp.float32)


def _shift_up(a, fill=0.0):
    pad = jnp.full((1, a.shape[1]), fill, a.dtype)
    return jnp.concatenate([a[1:], pad], axis=0)


def _shift_dn(a, fill=0.0):
    pad = jnp.full((1, a.shape[1]), fill, a.dtype)
    return jnp.concatenate([pad, a[:-1]], axis=0)


def _lshift_up(a, fill):
    pad = jnp.full((a.shape[0], 1), fill, a.dtype)
    return jnp.concatenate([a[:, 1:], pad], axis=1)


def _gat(x, Ws, Wd, a_src_flat, a_dst_flat, heads, ch, relu):
    xs = _bdott(x, Ws)
    xd = _bdott(x, Wd)
    rows = []
    for h in range(heads):
        sl = slice(h * ch, (h + 1) * ch)
        rows.append(jax.lax.dot_general(
            a_src_flat[:, sl], Ws[sl, :], (((1,), (0,)), ((), ())),
            preferred_element_type=jnp.float32))
        rows.append(jax.lax.dot_general(
            a_dst_flat[:, sl], Wd[sl, :], (((1,), (0,)), ((), ())),
            preferred_element_type=jnp.float32))
    wt = jnp.concatenate(rows, axis=0)
    sall = jax.lax.dot_general(wt, x, _DNT,
                               preferred_element_type=jnp.float32)
    hscale = 1.0 / heads
    acc = None
    for h in range(heads):
        ss = sall[2 * h:2 * h + 1, :]
        sd = sall[2 * h + 1:2 * h + 2, :]
        xsh = xs[:, h * ch:(h + 1) * ch]
        l1 = ss + _lshift_up(sd, _NEG)
        l2 = _lshift_up(ss, _NEG) + sd
        l3 = ss + sd
        m = jnp.max(ss) + jnp.max(sd)
        e1 = jnp.exp(l1 - m)
        e2 = jnp.exp(l2 - m)
        e3 = jnp.exp(l3 - m)
        z = hscale / (jnp.sum(e1) + jnp.sum(e2) + jnp.sum(e3))
        e1c = (e1 * z).reshape(-1, 1)
        e2c = (e2 * z).reshape(-1, 1)
        e3c = (e3 * z).reshape(-1, 1)
        out = e3c * xsh + _shift_dn(e1c * xsh) + e2c * _shift_up(xsh)
        acc = out if acc is None else acc + out
    if relu:
        acc = jnp.maximum(acc, 0.0)
    return acc


def _layernorm(x, g, b):
    m = jnp.mean(x, axis=1, keepdims=True)
    c = x - m
    v = jnp.mean(c * c, axis=1, keepdims=True)
    return c * jax.lax.rsqrt(v + 1e-5) * g + b


def _body(window_ref,
          e0Ws, e0Wd, e0as, e0ad,
          e1Ws, e1Wd, e1as, e1ad,
          qkvW, qkvb, projW, projb,
          f1W, f1b, f2W, f2b,
          n1g, n1b, n2g, n2b,
          d0Ws, d0Wd, d0as, d0ad,
          d1Ws, d1Wd, d1as, d1ad,
          clsW, clsb,
          out_ref, logits_ref):
    x = _gat(window_ref[...], e0Ws[...], e0Wd[...], e0as[...], e0ad[...],
             _NHEAD, _HID, relu=True)
    x = _gat(x, e1Ws[...], e1Wd[...], e1as[...], e1ad[...],
             _NHEAD, _HID, relu=True)

    res = x
    xn = _layernorm(x, n1g[...], n1b[...])
    qkv = _bdott(xn, qkvW[...]) + qkvb[...]
    scale = 1.0 / (_HD ** 0.5)
    head_outs = []
    for h in range(_NHEAD):
        qh = (qkv[:, h * _HD:(h + 1) * _HD] * scale).astype(jnp.bfloat16)
        kh = qkv[:, _HID + h * _HD:_HID + (h + 1) * _HD].astype(jnp.bfloat16)
        vh = qkv[:, 2 * _HID + h * _HD:2 * _HID + (h + 1) * _HD].astype(jnp.bfloat16)
        blocks = []
        for b in range(_T // _QBLK):
            qb = qh[b * _QBLK:(b + 1) * _QBLK]
            s = jax.lax.dot_general(qb, kh, _DNT,
                                    preferred_element_type=jnp.float32)
            mx = jnp.max(s, axis=1, keepdims=True)
            e = jnp.exp(s - mx)
            r = 1.0 / jnp.sum(e, axis=1, keepdims=True)
            ob = jnp.dot(e.astype(jnp.bfloat16), vh,
                         preferred_element_type=jnp.float32)
            blocks.append(ob * r)
        head_outs.append(jnp.concatenate(blocks, axis=0))
    o = jnp.concatenate(head_outs, axis=1)
    x = res + _bdott(o, projW[...]) + projb[...]
    res = x
    xn = _layernorm(x, n2g[...], n2b[...])
    f = _bdott(xn, f1W[...]) + f1b[...]
    f = 0.5 * f * (1.0 + jax.lax.erf(f * (2.0 ** -0.5)))
    x = res + _bdott(f, f2W[...]) + f2b[...]

    h_cls = jnp.mean(x, axis=0, keepdims=True)
    lg = jax.lax.dot_general(h_cls, clsW[...], _DNT,
                             preferred_element_type=jnp.float32) + clsb[...]
    logits_ref[...] = jnp.concatenate(
        [lg, jnp.zeros((1, 126), jnp.float32)], axis=1)

    x = _gat(x, d0Ws[...], d0Wd[...], d0as[...], d0ad[...],
             _NHEAD, _HID, relu=True)
    x = _gat(x, d1Ws[...], d1Wd[...], d1as[...], d1ad[...],
             1, _OUT_CH, relu=False)
    out_ref[...] = x.T


def kernel(window, enc0_Wsrc, enc0_Wdst, enc0_asrc, enc0_adst,
           enc1_Wsrc, enc1_Wdst, enc1_asrc, enc1_adst,
           qkv_W, qkv_b, proj_W, proj_b, ffn1_W, ffn1_b, ffn2_W, ffn2_b,
           norm1_g, norm1_b, norm2_g, norm2_b,
           dec0_Wsrc, dec0_Wdst, dec0_asrc, dec0_adst,
           dec1_Wsrc, dec1_Wdst, dec1_asrc, dec1_adst, cls_W, cls_b):
    f32 = jnp.float32
    operands = (
        window,
        enc0_Wsrc, enc0_Wdst,
        enc0_asrc.reshape(1, -1), enc0_adst.reshape(1, -1),
        enc1_Wsrc, enc1_Wdst,
        enc1_asrc.reshape(1, -1), enc1_adst.reshape(1, -1),
        qkv_W, qkv_b.reshape(1, -1), proj_W, proj_b.reshape(1, -1),
        ffn1_W, ffn1_b.reshape(1, -1), ffn2_W, ffn2_b.reshape(1, -1),
        norm1_g.reshape(1, -1), norm1_b.reshape(1, -1),
        norm2_g.reshape(1, -1), norm2_b.reshape(1, -1),
        dec0_Wsrc, dec0_Wdst,
        dec0_asrc.reshape(1, -1), dec0_adst.reshape(1, -1),
        dec1_Wsrc, dec1_Wdst,
        dec1_asrc.reshape(1, -1), dec1_adst.reshape(1, -1),
        cls_W, cls_b.reshape(1, -1),
    )
    out_t, logits_p = pl.pallas_call(
        _body,
        out_shape=(
            jax.ShapeDtypeStruct((_OUT_CH, _T), f32),
            jax.ShapeDtypeStruct((1, 128), f32),
        ),
    )(*operands)
    return (out_t, logits_p[0, :2])


# speedup vs baseline: 1.0266x; 1.0266x over previous
import jax
import jax.numpy as jnp
from jax.experimental import pallas as pl

_T = 2048
_HID = 256
_OUT_CH = 128
_NHEAD = 2
_HD = _HID // _NHEAD
_QBLK = 512
_NEG = -1e30

_DNT = (((1,), (1,)), ((), ()))


def _bdott(a, b):
    return jax.lax.dot_general(a.astype(jnp.bfloat16), b.astype(jnp.bfloat16),
                               _DNT, preferred_element_type=jnp.float32)


def _shift_up(a, fill=0.0):
    pad = jnp.full((1, a.shape[1]), fill, a.dtype)
    return jnp.concatenate([a[1:], pad], axis=0)


def _shift_dn(a, fill=0.0):
    pad = jnp.full((1, a.shape[1]), fill, a.dtype)
    return jnp.concatenate([pad, a[:-1]], axis=0)


def _lshift_up(a, fill):
    pad = jnp.full((a.shape[0], 1), fill, a.dtype)
    return jnp.concatenate([a[:, 1:], pad], axis=1)


def _gat(x, Ws, Wd, a_src_flat, a_dst_flat, heads, ch, relu):
    xs = _bdott(x, Ws)
    xd = _bdott(x, Wd)
    rows = []
    for h in range(heads):
        sl = slice(h * ch, (h + 1) * ch)
        rows.append(jax.lax.dot_general(
            a_src_flat[:, sl], Ws[sl, :], (((1,), (0,)), ((), ())),
            preferred_element_type=jnp.float32))
        rows.append(jax.lax.dot_general(
            a_dst_flat[:, sl], Wd[sl, :], (((1,), (0,)), ((), ())),
            preferred_element_type=jnp.float32))
    wt = jnp.concatenate(rows, axis=0)
    sall = jax.lax.dot_general(wt, x, _DNT,
                               preferred_element_type=jnp.float32)
    hscale = 1.0 / heads
    acc = None
    for h in range(heads):
        ss = sall[2 * h:2 * h + 1, :]
        sd = sall[2 * h + 1:2 * h + 2, :]
        xsh = xs[:, h * ch:(h + 1) * ch]
        l1 = ss + _lshift_up(sd, _NEG)
        l2 = _lshift_up(ss, _NEG) + sd
        l3 = ss + sd
        m = jnp.max(ss) + jnp.max(sd)
        e1 = jnp.exp(l1 - m)
        e2 = jnp.exp(l2 - m)
        e3 = jnp.exp(l3 - m)
        z = hscale / (jnp.sum(e1) + jnp.sum(e2) + jnp.sum(e3))
        e1c = (e1 * z).reshape(-1, 1)
        e2c = (e2 * z).reshape(-1, 1)
        e3c = (e3 * z).reshape(-1, 1)
        out = e3c * xsh + _shift_dn(e1c * xsh) + e2c * _shift_up(xsh)
        acc = out if acc is None else acc + out
    if relu:
        acc = jnp.maximum(acc, 0.0)
    return acc


def _layernorm(x, g, b):
    m = jnp.mean(x, axis=1, keepdims=True)
    c = x - m
    v = jnp.mean(c * c, axis=1, keepdims=True)
    return c * jax.lax.rsqrt(v + 1e-5) * g + b


def _body(window_ref,
          e0Ws, e0Wd, e0as, e0ad,
          e1Ws, e1Wd, e1as, e1ad,
          qkvW, qkvb, projW, projb,
          f1W, f1b, f2W, f2b,
          n1g, n1b, n2g, n2b,
          d0Ws, d0Wd, d0as, d0ad,
          d1Ws, d1Wd, d1as, d1ad,
          clsW, clsb,
          out_ref, logits_ref):
    x = _gat(window_ref[...], e0Ws[...], e0Wd[...], e0as[...], e0ad[...],
             _NHEAD, _HID, relu=True)
    x = _gat(x, e1Ws[...], e1Wd[...], e1as[...], e1ad[...],
             _NHEAD, _HID, relu=True)

    res = x
    xn = _layernorm(x, n1g[...], n1b[...])
    qkv = _bdott(xn, qkvW[...]) + qkvb[...]
    scale = 1.0 / (_HD ** 0.5)
    head_outs = []
    for h in range(_NHEAD):
        qh = (qkv[:, h * _HD:(h + 1) * _HD] * scale).astype(jnp.bfloat16)
        kh = qkv[:, _HID + h * _HD:_HID + (h + 1) * _HD].astype(jnp.bfloat16)
        vh = qkv[:, 2 * _HID + h * _HD:2 * _HID + (h + 1) * _HD].astype(jnp.bfloat16)
        blocks = []
        for b in range(_T // _QBLK):
            qb = qh[b * _QBLK:(b + 1) * _QBLK]
            s = jax.lax.dot_general(qb, kh, _DNT,
                                    preferred_element_type=jnp.float32)
            mx = jnp.max(s, axis=1, keepdims=True)
            e = jnp.exp(s - mx)
            r = 1.0 / jnp.sum(e, axis=1, keepdims=True)
            ob = jnp.dot(e.astype(jnp.bfloat16), vh,
                         preferred_element_type=jnp.float32)
            blocks.append(ob * r)
        head_outs.append(jnp.concatenate(blocks, axis=0))
    o = jnp.concatenate(head_outs, axis=1)
    x = res + _bdott(o, projW[...]) + projb[...]
    res = x
    xn = _layernorm(x, n2g[...], n2b[...])
    f = _bdott(xn, f1W[...]) + f1b[...]
    f = 0.5 * f * (1.0 + jax.lax.erf(f * (2.0 ** -0.5)))
    x = res + _bdott(f, f2W[...]) + f2b[...]

    h_cls = jnp.mean(x, axis=0, keepdims=True)
    lg = jax.lax.dot_general(h_cls, clsW[...], _DNT,
                             preferred_element_type=jnp.float32) + clsb[...]
    logits_ref[...] = jnp.concatenate(
        [lg, jnp.zeros((1, 126), jnp.float32)], axis=1)

    x = _gat(x, d0Ws[...], d0Wd[...], d0as[...], d0ad[...],
             _NHEAD, _HID, relu=True)
    x = _gat(x, d1Ws[...], d1Wd[...], d1as[...], d1ad[...],
             1, _OUT_CH, relu=False)
    out_ref[...] = x.T


def kernel(window, enc0_Wsrc, enc0_Wdst, enc0_asrc, enc0_adst,
           enc1_Wsrc, enc1_Wdst, enc1_asrc, enc1_adst,
           qkv_W, qkv_b, proj_W, proj_b, ffn1_W, ffn1_b, ffn2_W, ffn2_b,
           norm1_g, norm1_b, norm2_g, norm2_b,
           dec0_Wsrc, dec0_Wdst, dec0_asrc, dec0_adst,
           dec1_Wsrc, dec1_Wdst, dec1_asrc, dec1_adst, cls_W, cls_b):
    f32 = jnp.float32
    operands = (
        window,
        enc0_Wsrc, enc0_Wdst,
        enc0_asrc.reshape(1, -1), enc0_adst.reshape(1, -1),
        enc1_Wsrc, enc1_Wdst,
        enc1_asrc.reshape(1, -1), enc1_adst.reshape(1, -1),
        qkv_W, qkv_b.reshape(1, -1), proj_W, proj_b.reshape(1, -1),
        ffn1_W, ffn1_b.reshape(1, -1), ffn2_W, ffn2_b.reshape(1, -1),
        norm1_g.reshape(1, -1), norm1_b.reshape(1, -1),
        norm2_g.reshape(1, -1), norm2_b.reshape(1, -1),
        dec0_Wsrc, dec0_Wdst,
        dec0_asrc.reshape(1, -1), dec0_adst.reshape(1, -1),
        dec1_Wsrc, dec1_Wdst,
        dec1_asrc.reshape(1, -1), dec1_adst.reshape(1, -1),
        cls_W, cls_b.reshape(1, -1),
    )
    out_t, logits_p = pl.pallas_call(
        _body,
        out_shape=(
            jax.ShapeDtypeStruct((_OUT_CH, _T), f32),
            jax.ShapeDtypeStruct((1, 128), f32),
        ),
    )(*operands)
    return (out_t, logits_p[0, :2])


# speedup vs baseline: 1.0411x; 1.0141x over previous
import jax
import jax.numpy as jnp
from jax.experimental import pallas as pl

_T = 2048
_HID = 256
_OUT_CH = 128
_NHEAD = 2
_HD = _HID // _NHEAD
_QBLK = 512
_NEG = -1e30

_DNT = (((1,), (1,)), ((), ()))


def _bdott(a, b):
    return jax.lax.dot_general(a.astype(jnp.bfloat16), b.astype(jnp.bfloat16),
                               _DNT, preferred_element_type=jnp.float32)


def _shift_up(a, fill=0.0):
    pad = jnp.full((1, a.shape[1]), fill, a.dtype)
    return jnp.concatenate([a[1:], pad], axis=0)


def _shift_dn(a, fill=0.0):
    pad = jnp.full((1, a.shape[1]), fill, a.dtype)
    return jnp.concatenate([pad, a[:-1]], axis=0)


def _lshift_up(a, fill):
    pad = jnp.full((a.shape[0], 1), fill, a.dtype)
    return jnp.concatenate([a[:, 1:], pad], axis=1)


def _lshift_dn(a, fill):
    pad = jnp.full((a.shape[0], 1), fill, a.dtype)
    return jnp.concatenate([pad, a[:, :-1]], axis=1)


def _gat_last_T(x, Ws, Wd, a_src_flat, a_dst_flat):
    xsT = _bdott(Ws, x)
    ws = jax.lax.dot_general(a_src_flat, Ws, (((1,), (0,)), ((), ())),
                             preferred_element_type=jnp.float32)
    wd = jax.lax.dot_general(a_dst_flat, Wd, (((1,), (0,)), ((), ())),
                             preferred_element_type=jnp.float32)
    wt = jnp.concatenate([ws, wd], axis=0)
    sall = jax.lax.dot_general(wt, x, _DNT,
                               preferred_element_type=jnp.float32)
    ss = sall[0:1, :]
    sd = sall[1:2, :]
    l1 = ss + _lshift_up(sd, _NEG)
    l2 = _lshift_up(ss, _NEG) + sd
    l3 = ss + sd
    m = jnp.max(ss) + jnp.max(sd)
    e1 = jnp.exp(l1 - m)
    e2 = jnp.exp(l2 - m)
    e3 = jnp.exp(l3 - m)
    z = 1.0 / (jnp.sum(e1) + jnp.sum(e2) + jnp.sum(e3))
    e1 = e1 * z
    e2 = e2 * z
    e3 = e3 * z
    return e3 * xsT + _lshift_dn(e1 * xsT, 0.0) + e2 * _lshift_up(xsT, 0.0)


def _gat(x, Ws, Wd, a_src_flat, a_dst_flat, heads, ch, relu):
    xs = _bdott(x, Ws)
    xd = _bdott(x, Wd)
    rows = []
    for h in range(heads):
        sl = slice(h * ch, (h + 1) * ch)
        rows.append(jax.lax.dot_general(
            a_src_flat[:, sl], Ws[sl, :], (((1,), (0,)), ((), ())),
            preferred_element_type=jnp.float32))
        rows.append(jax.lax.dot_general(
            a_dst_flat[:, sl], Wd[sl, :], (((1,), (0,)), ((), ())),
            preferred_element_type=jnp.float32))
    wt = jnp.concatenate(rows, axis=0)
    sall = jax.lax.dot_general(wt, x, _DNT,
                               preferred_element_type=jnp.float32)
    hscale = 1.0 / heads
    acc = None
    for h in range(heads):
        ss = sall[2 * h:2 * h + 1, :]
        sd = sall[2 * h + 1:2 * h + 2, :]
        xsh = xs[:, h * ch:(h + 1) * ch]
        l1 = ss + _lshift_up(sd, _NEG)
        l2 = _lshift_up(ss, _NEG) + sd
        l3 = ss + sd
        m = jnp.max(ss) + jnp.max(sd)
        e1 = jnp.exp(l1 - m)
        e2 = jnp.exp(l2 - m)
        e3 = jnp.exp(l3 - m)
        z = hscale / (jnp.sum(e1) + jnp.sum(e2) + jnp.sum(e3))
        e1c = (e1 * z).reshape(-1, 1)
        e2c = (e2 * z).reshape(-1, 1)
        e3c = (e3 * z).reshape(-1, 1)
        out = e3c * xsh + _shift_dn(e1c * xsh) + e2c * _shift_up(xsh)
        acc = out if acc is None else acc + out
    if relu:
        acc = jnp.maximum(acc, 0.0)
    return acc


def _layernorm(x, g, b):
    m = jnp.mean(x, axis=1, keepdims=True)
    c = x - m
    v = jnp.mean(c * c, axis=1, keepdims=True)
    return c * jax.lax.rsqrt(v + 1e-5) * g + b


def _body(window_ref,
          e0Ws, e0Wd, e0as, e0ad,
          e1Ws, e1Wd, e1as, e1ad,
          qkvW, qkvb, projW, projb,
          f1W, f1b, f2W, f2b,
          n1g, n1b, n2g, n2b,
          d0Ws, d0Wd, d0as, d0ad,
          d1Ws, d1Wd, d1as, d1ad,
          clsW, clsb,
          out_ref, logits_ref):
    x = _gat(window_ref[...], e0Ws[...], e0Wd[...], e0as[...], e0ad[...],
             _NHEAD, _HID, relu=True)
    x = _gat(x, e1Ws[...], e1Wd[...], e1as[...], e1ad[...],
             _NHEAD, _HID, relu=True)

    res = x
    xn = _layernorm(x, n1g[...], n1b[...])
    qkv = _bdott(xn, qkvW[...]) + qkvb[...]
    scale = 1.0 / (_HD ** 0.5)
    head_outs = []
    for h in range(_NHEAD):
        qh = (qkv[:, h * _HD:(h + 1) * _HD] * scale).astype(jnp.bfloat16)
        kh = qkv[:, _HID + h * _HD:_HID + (h + 1) * _HD].astype(jnp.bfloat16)
        vh = qkv[:, 2 * _HID + h * _HD:2 * _HID + (h + 1) * _HD].astype(jnp.bfloat16)
        blocks = []
        for b in range(_T // _QBLK):
            qb = qh[b * _QBLK:(b + 1) * _QBLK]
            s = jax.lax.dot_general(qb, kh, _DNT,
                                    preferred_element_type=jnp.float32)
            mx = jnp.max(s, axis=1, keepdims=True)
            e = jnp.exp(s - mx)
            r = 1.0 / jnp.sum(e, axis=1, keepdims=True)
            ob = jnp.dot(e.astype(jnp.bfloat16), vh,
                         preferred_element_type=jnp.float32)
            blocks.append(ob * r)
        head_outs.append(jnp.concatenate(blocks, axis=0))
    o = jnp.concatenate(head_outs, axis=1)
    x = res + _bdott(o, projW[...]) + projb[...]
    res = x
    xn = _layernorm(x, n2g[...], n2b[...])
    f = _bdott(xn, f1W[...]) + f1b[...]
    f = 0.5 * f * (1.0 + jax.lax.erf(f * (2.0 ** -0.5)))
    x = res + _bdott(f, f2W[...]) + f2b[...]

    h_cls = jnp.mean(x, axis=0, keepdims=True)
    lg = jax.lax.dot_general(h_cls, clsW[...], _DNT,
                             preferred_element_type=jnp.float32) + clsb[...]
    logits_ref[...] = jnp.concatenate(
        [lg, jnp.zeros((1, 126), jnp.float32)], axis=1)

    x = _gat(x, d0Ws[...], d0Wd[...], d0as[...], d0ad[...],
             _NHEAD, _HID, relu=True)
    out_ref[...] = _gat_last_T(x, d1Ws[...], d1Wd[...], d1as[...], d1ad[...])


def kernel(window, enc0_Wsrc, enc0_Wdst, enc0_asrc, enc0_adst,
           enc1_Wsrc, enc1_Wdst, enc1_asrc, enc1_adst,
           qkv_W, qkv_b, proj_W, proj_b, ffn1_W, ffn1_b, ffn2_W, ffn2_b,
           norm1_g, norm1_b, norm2_g, norm2_b,
           dec0_Wsrc, dec0_Wdst, dec0_asrc, dec0_adst,
           dec1_Wsrc, dec1_Wdst, dec1_asrc, dec1_adst, cls_W, cls_b):
    f32 = jnp.float32
    operands = (
        window,
        enc0_Wsrc, enc0_Wdst,
        enc0_asrc.reshape(1, -1), enc0_adst.reshape(1, -1),
        enc1_Wsrc, enc1_Wdst,
        enc1_asrc.reshape(1, -1), enc1_adst.reshape(1, -1),
        qkv_W, qkv_b.reshape(1, -1), proj_W, proj_b.reshape(1, -1),
        ffn1_W, ffn1_b.reshape(1, -1), ffn2_W, ffn2_b.reshape(1, -1),
        norm1_g.reshape(1, -1), norm1_b.reshape(1, -1),
        norm2_g.reshape(1, -1), norm2_b.reshape(1, -1),
        dec0_Wsrc, dec0_Wdst,
        dec0_asrc.reshape(1, -1), dec0_adst.reshape(1, -1),
        dec1_Wsrc, dec1_Wdst,
        dec1_asrc.reshape(1, -1), dec1_adst.reshape(1, -1),
        cls_W, cls_b.reshape(1, -1),
    )
    out_t, logits_p = pl.pallas_call(
        _body,
        out_shape=(
            jax.ShapeDtypeStruct((_OUT_CH, _T), f32),
            jax.ShapeDtypeStruct((1, 128), f32),
        ),
    )(*operands)
    return (out_t, logits_p[0, :2])
